# baseline (speedup 1.0000x reference)
import functools

import jax
import jax.numpy as jnp
from jax import lax
from jax.experimental import pallas as pl
from jax.experimental.pallas import tpu as pltpu
from jax.experimental.pallas import tpu_sc as plsc

EMBED = 256
N = 8192
BM = 2048
BN = 512
GI = N // BM
GJ = N // BN
GT = GJ // 2 + 1
NLANE = 128
NG = BN // NLANE
FBM = 1024


def _epilogue(mm_ref, z2b, w2b, smin, sarg, jb):
    rm = smin[...]
    ra = sarg[...]
    for g in range(NG):
        sl = slice(g * NLANE, (g + 1) * NLANE)
        dg = (z2b - mm_ref[:, sl]) + w2b[:, sl]
        sf = (jb * NG + g).astype(jnp.float32)
        better = dg < rm
        rm = jnp.where(better, dg, rm)
        ra = jnp.where(better, sf, ra)
    smin[...] = rm
    sarg[...] = ra


def _sweep_body(z_ref, wa_ref, wb_ref, z2_ref, w2a_ref, w2b_ref,
                rmin_ref, rarg_ref, mmA, mmB, smin, sarg):
    t = pl.program_id(1)
    zb = z_ref[...]
    z2b = z2_ref[...]

    @pl.when(t == 0)
    def _init():
        smin[...] = jnp.full((BM, NLANE), jnp.inf, jnp.float32)
        sarg[...] = jnp.zeros((BM, NLANE), jnp.float32)

    @pl.when(t > 0)
    def _epi_prev():
        _epilogue(mmB, z2b, w2b_ref[...], smin, sarg, 2 * t - 1)

    mmA[...] = jax.lax.dot_general(
        zb, wa_ref[...] * 2.0, (((1,), (1,)), ((), ())),
        preferred_element_type=jnp.float32)

    _epilogue(mmA, z2b, w2a_ref[...], smin, sarg, 2 * t)

    mmB[...] = jax.lax.dot_general(
        zb, wb_ref[...] * 2.0, (((1,), (1,)), ((), ())),
        preferred_element_type=jnp.float32)

    @pl.when(t == GT - 1)
    def _flush():
        rmin_ref[...] = smin[...]
        rarg_ref[...] = sarg[...]


def _final_body(rmin_ref, rarg_ref, idx_ref):
    rm = rmin_ref[...]
    col = rarg_ref[...] * 128.0 + jax.lax.broadcasted_iota(
        jnp.int32, (FBM, NLANE), 1).astype(jnp.float32)
    gmin = jnp.min(rm, axis=1, keepdims=True)
    cand = jnp.where(rm == gmin, col, jnp.float32(2**24))
    idx_ref[...] = jnp.min(cand, axis=1, keepdims=True).astype(jnp.int32)


@functools.partial(jax.jit)
def _encode(z, W, z2, w2):
    clamp = GJ - 1
    rmin, rarg = pl.pallas_call(
        _sweep_body,
        grid=(GI, GT),
        in_specs=[
            pl.BlockSpec((BM, EMBED), lambda i, t: (i, 0)),
            pl.BlockSpec((BN, EMBED), lambda i, t: (jnp.minimum(2 * t, clamp), 0)),
            pl.BlockSpec((BN, EMBED), lambda i, t: (jnp.minimum(2 * t + 1, clamp), 0)),
            pl.BlockSpec((BM, 1), lambda i, t: (i, 0)),
            pl.BlockSpec((1, BN), lambda i, t: (0, jnp.minimum(2 * t, clamp))),
            pl.BlockSpec((1, BN), lambda i, t: (0, jnp.maximum(2 * t - 1, 0))),
        ],
        out_specs=[
            pl.BlockSpec((BM, NLANE), lambda i, t: (i, 0)),
            pl.BlockSpec((BM, NLANE), lambda i, t: (i, 0)),
        ],
        out_shape=[
            jax.ShapeDtypeStruct((N, NLANE), jnp.float32),
            jax.ShapeDtypeStruct((N, NLANE), jnp.float32),
        ],
        scratch_shapes=[
            pltpu.VMEM((BM, BN), jnp.float32),
            pltpu.VMEM((BM, BN), jnp.float32),
            pltpu.VMEM((BM, NLANE), jnp.float32),
            pltpu.VMEM((BM, NLANE), jnp.float32),
        ],
        compiler_params=pltpu.CompilerParams(
            dimension_semantics=("parallel", "arbitrary")),
    )(z, W, W, z2, w2, w2)
    idx = pl.pallas_call(
        _final_body,
        grid=(N // FBM,),
        in_specs=[
            pl.BlockSpec((FBM, NLANE), lambda i: (i, 0)),
            pl.BlockSpec((FBM, NLANE), lambda i: (i, 0)),
        ],
        out_specs=pl.BlockSpec((FBM, 1), lambda i: (i, 0)),
        out_shape=jax.ShapeDtypeStruct((N, 1), jnp.int32),
        compiler_params=pltpu.CompilerParams(
            dimension_semantics=("parallel",)),
    )(rmin, rarg)
    return idx


_sc_info = plsc.get_sparse_core_info()
_NC = _sc_info.num_cores
_NS = _sc_info.num_subcores
NW = _NC * _NS
ROWS_W = N // NW
CH = 128
NCH = ROWS_W // CH
NK = EMBED // 16

_sc_mesh = plsc.VectorSubcoreMesh(core_axis_name="c", subcore_axis_name="s")


@functools.partial(
    pl.kernel,
    mesh=_sc_mesh,
    out_type=[
        jax.ShapeDtypeStruct((N, EMBED), jnp.float32),
        jax.ShapeDtypeStruct((NW, 16), jnp.float32),
    ],
    scratch_types=[
        pltpu.VMEM((CH,), jnp.int32),
        pltpu.VMEM((CH, EMBED), jnp.float32),
        pltpu.VMEM((CH, EMBED), jnp.float32),
        pltpu.VMEM((16,), jnp.float32),
        pltpu.SemaphoreType.DMA,
    ],
)
def _gather_st_loss(idx_hbm, z_hbm, w_hbm, zqst_hbm, part_hbm,
                    idx_v, rows_v, z_v, acc_v, sem):
    wid = lax.axis_index("s") * _NC + lax.axis_index("c")
    acc = jnp.zeros((16,), jnp.float32)
    for c in range(NCH):
        rb = wid * ROWS_W + c * CH
        pltpu.sync_copy(idx_hbm.at[pl.ds(rb, CH)], idx_v)
        pltpu.async_copy(w_hbm.at[idx_v], rows_v, sem).wait()
        pltpu.sync_copy(z_hbm.at[pl.ds(rb, CH)], z_v)

        def row_body(r, acc):
            racc = jnp.zeros((16,), jnp.float32)
            for k in range(NK):
                sl = pl.ds(k * 16, 16)
                t = rows_v[r, sl] - z_v[r, sl]
                rows_v[r, sl] = z_v[r, sl] + t
                racc = racc + t * t
            return acc + racc

        acc = lax.fori_loop(0, CH, row_body, acc)
        pltpu.sync_copy(rows_v, zqst_hbm.at[pl.ds(rb, CH)])
    acc_v[...] = acc
    pltpu.sync_copy(acc_v, part_hbm.at[wid])


def kernel(z, W):
    z2 = jnp.sum(z ** 2, axis=1, keepdims=True)
    w2 = jnp.sum(W ** 2, axis=1)[None, :]
    idx = _encode(z, W, z2, w2).reshape(N)
    z_q_st, parts = _gather_st_loss(idx, z, W)
    mse = parts.sum() / jnp.float32(N * EMBED)
    vq_loss = mse + 0.25 * mse
    return (z_q_st, vq_loss)


# speedup vs baseline: 1.1131x; 1.1131x over previous
import functools

import jax
import jax.numpy as jnp
from jax import lax
from jax.experimental import pallas as pl
from jax.experimental.pallas import tpu as pltpu
from jax.experimental.pallas import tpu_sc as plsc

EMBED = 256
N = 8192
BM = 2048
BN = 512
GI = N // BM
GJ = N // BN
GT = GJ // 2 + 1
NLANE = 128
NG = BN // NLANE
FBM = 1024


def _epilogue(mm_ref, z2b, w2b, rm, ra, jb):
    for g in range(NG):
        sl = slice(g * NLANE, (g + 1) * NLANE)
        dg = (z2b - mm_ref[:, sl]) + w2b[:, sl]
        sf = (jb * NG + g).astype(jnp.float32)
        better = dg < rm
        rm = jnp.where(better, dg, rm)
        ra = jnp.where(better, sf, ra)
    return rm, ra


def _sweep_body(z_ref, wa_ref, wb_ref, z2_ref, w2a_ref, w2b_ref,
                rmin_ref, rarg_ref, mmA, mmB, smin, sarg):
    t = pl.program_id(1)
    zb = z_ref[...]
    z2b = z2_ref[...]

    @pl.when(t == 0)
    def _init():
        smin[...] = jnp.full((BM, NLANE), jnp.inf, jnp.float32)
        sarg[...] = jnp.zeros((BM, NLANE), jnp.float32)

    rm = smin[...]
    ra = sarg[...]
    rm, ra = _epilogue(mmB, z2b, w2b_ref[...], rm, ra, 2 * t - 1)

    mmA[...] = jax.lax.dot_general(
        zb, wa_ref[...] * 2.0, (((1,), (1,)), ((), ())),
        preferred_element_type=jnp.float32)

    rm, ra = _epilogue(mmA, z2b, w2a_ref[...], rm, ra, 2 * t)
    smin[...] = rm
    sarg[...] = ra

    mmB[...] = jax.lax.dot_general(
        zb, wb_ref[...] * 2.0, (((1,), (1,)), ((), ())),
        preferred_element_type=jnp.float32)

    @pl.when(t == GT - 1)
    def _flush():
        rmin_ref[...] = rm
        rarg_ref[...] = ra


def _final_body(rmin_ref, rarg_ref, idx_ref):
    rm = rmin_ref[...]
    col = rarg_ref[...] * 128.0 + jax.lax.broadcasted_iota(
        jnp.int32, (FBM, NLANE), 1).astype(jnp.float32)
    gmin = jnp.min(rm, axis=1, keepdims=True)
    cand = jnp.where(rm == gmin, col, jnp.float32(2**24))
    idx_ref[...] = jnp.min(cand, axis=1, keepdims=True).astype(jnp.int32)


@functools.partial(jax.jit)
def _encode(z, W, z2, w2):
    clamp = GJ - 1
    w2aug = jnp.concatenate(
        [w2, jnp.full((1, BN), jnp.inf, jnp.float32)], axis=1)
    rmin, rarg = pl.pallas_call(
        _sweep_body,
        grid=(GI, GT),
        in_specs=[
            pl.BlockSpec((BM, EMBED), lambda i, t: (i, 0)),
            pl.BlockSpec((BN, EMBED), lambda i, t: (jnp.minimum(2 * t, clamp), 0)),
            pl.BlockSpec((BN, EMBED), lambda i, t: (jnp.minimum(2 * t + 1, clamp), 0)),
            pl.BlockSpec((BM, 1), lambda i, t: (i, 0)),
            pl.BlockSpec((1, BN), lambda i, t: (0, jnp.minimum(2 * t, clamp))),
            pl.BlockSpec((1, BN), lambda i, t: (0, jnp.where(t == 0, GJ, 2 * t - 1))),
        ],
        out_specs=[
            pl.BlockSpec((BM, NLANE), lambda i, t: (i, 0)),
            pl.BlockSpec((BM, NLANE), lambda i, t: (i, 0)),
        ],
        out_shape=[
            jax.ShapeDtypeStruct((N, NLANE), jnp.float32),
            jax.ShapeDtypeStruct((N, NLANE), jnp.float32),
        ],
        scratch_shapes=[
            pltpu.VMEM((BM, BN), jnp.float32),
            pltpu.VMEM((BM, BN), jnp.float32),
            pltpu.VMEM((BM, NLANE), jnp.float32),
            pltpu.VMEM((BM, NLANE), jnp.float32),
        ],
        compiler_params=pltpu.CompilerParams(
            dimension_semantics=("parallel", "arbitrary")),
    )(z, W, W, z2, w2, w2aug)
    idx = pl.pallas_call(
        _final_body,
        grid=(N // FBM,),
        in_specs=[
            pl.BlockSpec((FBM, NLANE), lambda i: (i, 0)),
            pl.BlockSpec((FBM, NLANE), lambda i: (i, 0)),
        ],
        out_specs=pl.BlockSpec((FBM, 1), lambda i: (i, 0)),
        out_shape=jax.ShapeDtypeStruct((N, 1), jnp.int32),
        compiler_params=pltpu.CompilerParams(
            dimension_semantics=("parallel",)),
    )(rmin, rarg)
    return idx


_sc_info = plsc.get_sparse_core_info()
_NC = _sc_info.num_cores
_NS = _sc_info.num_subcores
NW = _NC * _NS
ROWS_W = N // NW
CH = 128
NCH = ROWS_W // CH
NK = EMBED // 16

_sc_mesh = plsc.VectorSubcoreMesh(core_axis_name="c", subcore_axis_name="s")


@functools.partial(
    pl.kernel,
    mesh=_sc_mesh,
    out_type=[
        jax.ShapeDtypeStruct((N, EMBED), jnp.float32),
        jax.ShapeDtypeStruct((NW, 16), jnp.float32),
    ],
    scratch_types=[
        pltpu.VMEM((CH,), jnp.int32),
        pltpu.VMEM((CH, EMBED), jnp.float32),
        pltpu.VMEM((CH, EMBED), jnp.float32),
        pltpu.VMEM((16,), jnp.float32),
        pltpu.SemaphoreType.DMA,
    ],
)
def _gather_st_loss(idx_hbm, z_hbm, w_hbm, zqst_hbm, part_hbm,
                    idx_v, rows_v, z_v, acc_v, sem):
    wid = lax.axis_index("s") * _NC + lax.axis_index("c")
    acc = jnp.zeros((16,), jnp.float32)
    for c in range(NCH):
        rb = wid * ROWS_W + c * CH
        pltpu.sync_copy(idx_hbm.at[pl.ds(rb, CH)], idx_v)
        pltpu.async_copy(w_hbm.at[idx_v], rows_v, sem).wait()
        pltpu.sync_copy(z_hbm.at[pl.ds(rb, CH)], z_v)

        def row_body(r, acc):
            racc = jnp.zeros((16,), jnp.float32)
            for k in range(NK):
                sl = pl.ds(k * 16, 16)
                t = rows_v[r, sl] - z_v[r, sl]
                rows_v[r, sl] = z_v[r, sl] + t
                racc = racc + t * t
            return acc + racc

        acc = lax.fori_loop(0, CH, row_body, acc)
        pltpu.sync_copy(rows_v, zqst_hbm.at[pl.ds(rb, CH)])
    acc_v[...] = acc
    pltpu.sync_copy(acc_v, part_hbm.at[wid])


def kernel(z, W):
    z2 = jnp.sum(z ** 2, axis=1, keepdims=True)
    w2 = jnp.sum(W ** 2, axis=1)[None, :]
    idx = _encode(z, W, z2, w2).reshape(N)
    z_q_st, parts = _gather_st_loss(idx, z, W)
    mse = parts.sum() / jnp.float32(N * EMBED)
    vq_loss = mse + 0.25 * mse
    return (z_q_st, vq_loss)


# speedup vs baseline: 1.1410x; 1.0251x over previous
import functools

import jax
import jax.numpy as jnp
from jax import lax
from jax.experimental import pallas as pl
from jax.experimental.pallas import tpu as pltpu
from jax.experimental.pallas import tpu_sc as plsc

EMBED = 256
N = 8192
BM = 2048
BN = 512
GI = N // BM
GJ = N // BN
GT = GJ // 2 + 1
NLANE = 128
NG = BN // NLANE
FBM = 1024


def _epilogue(mm_ref, z2b, w2b, rm, ra, jb):
    for g in range(NG):
        sl = slice(g * NLANE, (g + 1) * NLANE)
        dg = (z2b - mm_ref[:, sl]) + w2b[:, sl]
        sf = (jb * NG + g).astype(jnp.float32)
        better = dg < rm
        rm = jnp.where(better, dg, rm)
        ra = jnp.where(better, sf, ra)
    return rm, ra


def _sweep_body(z_ref, wa_ref, wb_ref, z2_ref, w2a_ref, w2b_ref,
                rmin_ref, rarg_ref, mmA, mmB, smin, sarg):
    t = pl.program_id(1)
    zb = z_ref[...]
    z2b = z2_ref[...]

    @pl.when(t == 0)
    def _init():
        smin[...] = jnp.full((BM, NLANE), jnp.inf, jnp.float32)
        sarg[...] = jnp.zeros((BM, NLANE), jnp.float32)

    rm = smin[...]
    ra = sarg[...]
    rm, ra = _epilogue(mmB, z2b, w2b_ref[...], rm, ra, 2 * t - 1)

    mmA[...] = jax.lax.dot_general(
        zb, wa_ref[...] * 2.0, (((1,), (1,)), ((), ())),
        preferred_element_type=jnp.float32)

    rm, ra = _epilogue(mmA, z2b, w2a_ref[...], rm, ra, 2 * t)
    smin[...] = rm
    sarg[...] = ra

    mmB[...] = jax.lax.dot_general(
        zb, wb_ref[...] * 2.0, (((1,), (1,)), ((), ())),
        preferred_element_type=jnp.float32)

    @pl.when(t == GT - 1)
    def _flush():
        rmin_ref[...] = rm
        rarg_ref[...] = ra


def _final_body(rmin_ref, rarg_ref, idx_ref):
    rm = rmin_ref[...]
    col = rarg_ref[...] * 128.0 + jax.lax.broadcasted_iota(
        jnp.int32, (FBM, NLANE), 1).astype(jnp.float32)
    gmin = jnp.min(rm, axis=1, keepdims=True)
    cand = jnp.where(rm == gmin, col, jnp.float32(2**24))
    idx_ref[...] = jnp.min(cand, axis=1, keepdims=True).astype(jnp.int32)


@functools.partial(jax.jit)
def _encode(z, W, z2, w2):
    clamp = GJ - 1
    w2aug = jnp.concatenate(
        [w2, jnp.full((1, BN), jnp.inf, jnp.float32)], axis=1)
    rmin, rarg = pl.pallas_call(
        _sweep_body,
        grid=(GI, GT),
        in_specs=[
            pl.BlockSpec((BM, EMBED), lambda i, t: (i, 0)),
            pl.BlockSpec((BN, EMBED), lambda i, t: (jnp.minimum(2 * t, clamp), 0)),
            pl.BlockSpec((BN, EMBED), lambda i, t: (jnp.minimum(2 * t + 1, clamp), 0)),
            pl.BlockSpec((BM, 1), lambda i, t: (i, 0)),
            pl.BlockSpec((1, BN), lambda i, t: (0, jnp.minimum(2 * t, clamp))),
            pl.BlockSpec((1, BN), lambda i, t: (0, jnp.where(t == 0, GJ, 2 * t - 1))),
        ],
        out_specs=[
            pl.BlockSpec((BM, NLANE), lambda i, t: (i, 0)),
            pl.BlockSpec((BM, NLANE), lambda i, t: (i, 0)),
        ],
        out_shape=[
            jax.ShapeDtypeStruct((N, NLANE), jnp.float32),
            jax.ShapeDtypeStruct((N, NLANE), jnp.float32),
        ],
        scratch_shapes=[
            pltpu.VMEM((BM, BN), jnp.float32),
            pltpu.VMEM((BM, BN), jnp.float32),
            pltpu.VMEM((BM, NLANE), jnp.float32),
            pltpu.VMEM((BM, NLANE), jnp.float32),
        ],
        compiler_params=pltpu.CompilerParams(
            dimension_semantics=("parallel", "arbitrary")),
    )(z, W, W, z2, w2, w2aug)
    idx = pl.pallas_call(
        _final_body,
        grid=(N // FBM,),
        in_specs=[
            pl.BlockSpec((FBM, NLANE), lambda i: (i, 0)),
            pl.BlockSpec((FBM, NLANE), lambda i: (i, 0)),
        ],
        out_specs=pl.BlockSpec((FBM, 1), lambda i: (i, 0)),
        out_shape=jax.ShapeDtypeStruct((N, 1), jnp.int32),
        compiler_params=pltpu.CompilerParams(
            dimension_semantics=("parallel",)),
    )(rmin, rarg)
    return idx


_sc_info = plsc.get_sparse_core_info()
_NC = _sc_info.num_cores
_NS = _sc_info.num_subcores
NW = _NC * _NS
ROWS_W = N // NW
CH = 64
NCH = ROWS_W // CH
NK = EMBED // 16

_sc_mesh = plsc.VectorSubcoreMesh(core_axis_name="c", subcore_axis_name="s")


@functools.partial(
    pl.kernel,
    mesh=_sc_mesh,
    out_type=[
        jax.ShapeDtypeStruct((N, EMBED), jnp.float32),
        jax.ShapeDtypeStruct((NW, 16), jnp.float32),
    ],
    scratch_types=[
        pltpu.VMEM((CH,), jnp.int32),
        pltpu.VMEM((CH,), jnp.int32),
        pltpu.VMEM((CH, EMBED), jnp.float32),
        pltpu.VMEM((CH, EMBED), jnp.float32),
        pltpu.VMEM((CH, EMBED), jnp.float32),
        pltpu.VMEM((CH, EMBED), jnp.float32),
        pltpu.VMEM((16,), jnp.float32),
        pltpu.SemaphoreType.DMA,
        pltpu.SemaphoreType.DMA,
        pltpu.SemaphoreType.DMA,
    ],
)
def _gather_st_loss(idx_hbm, z_hbm, w_hbm, zqst_hbm, part_hbm,
                    idx0, idx1, rows0, rows1, z0, z1, acc_v,
                    gsem, zsem, osem):
    wid = lax.axis_index("s") * _NC + lax.axis_index("c")
    base = wid * ROWS_W
    bufs = ((idx0, rows0, z0), (idx1, rows1, z1))

    def start(c, slot):
        idxb, rowsb, zb = bufs[slot]
        rb = base + c * CH
        pltpu.sync_copy(idx_hbm.at[pl.ds(rb, CH)], idxb)
        g = pltpu.async_copy(w_hbm.at[idxb], rowsb, gsem)
        zc = pltpu.async_copy(z_hbm.at[pl.ds(rb, CH)], zb, zsem)
        return g, zc

    acc = jnp.zeros((16,), jnp.float32)
    inflight = {0: start(0, 0)}
    wb = {}
    for c in range(NCH):
        slot = c % 2
        if c >= 1 and (c - 1) in wb:
            wb[c - 1].wait()
        if c + 1 < NCH:
            inflight[c + 1] = start(c + 1, (c + 1) % 2)
        g, zc = inflight.pop(c)
        g.wait()
        zc.wait()
        _, rowsb, zb = bufs[slot]

        def row_body(r, acc, rowsb=rowsb, zb=zb):
            racc = jnp.zeros((16,), jnp.float32)
            for k in range(NK):
                sl = pl.ds(k * 16, 16)
                t = rowsb[r, sl] - zb[r, sl]
                rowsb[r, sl] = zb[r, sl] + t
                racc = racc + t * t
            return acc + racc

        acc = lax.fori_loop(0, CH, row_body, acc)
        wb[c] = pltpu.async_copy(
            rowsb, zqst_hbm.at[pl.ds(base + c * CH, CH)], osem)
    wb[NCH - 1].wait()
    acc_v[...] = acc
    pltpu.sync_copy(acc_v, part_hbm.at[wid])


def kernel(z, W):
    z2 = jnp.sum(z ** 2, axis=1, keepdims=True)
    w2 = jnp.sum(W ** 2, axis=1)[None, :]
    idx = _encode(z, W, z2, w2).reshape(N)
    z_q_st, parts = _gather_st_loss(idx, z, W)
    mse = parts.sum() / jnp.float32(N * EMBED)
    vq_loss = mse + 0.25 * mse
    return (z_q_st, vq_loss)


# speedup vs baseline: 1.1530x; 1.0105x over previous
import functools

import jax
import jax.numpy as jnp
from jax import lax
from jax.experimental import pallas as pl
from jax.experimental.pallas import tpu as pltpu
from jax.experimental.pallas import tpu_sc as plsc

EMBED = 256
N = 8192
BM = 2048
BN = 1024
GI = N // BM
GJ = N // BN
GT = GJ // 2 + 1
NLANE = 128
NG = BN // NLANE
FBM = 1024


def _epilogue(mm_ref, z2b, w2b, rm, ra, jb):
    for g in range(NG):
        sl = slice(g * NLANE, (g + 1) * NLANE)
        dg = (z2b - mm_ref[:, sl]) + w2b[:, sl]
        sf = (jb * NG + g).astype(jnp.float32)
        better = dg < rm
        rm = jnp.where(better, dg, rm)
        ra = jnp.where(better, sf, ra)
    return rm, ra


def _sweep_body(z_ref, wa_ref, wb_ref, z2_ref, w2a_ref, w2b_ref,
                rmin_ref, rarg_ref, mmA, mmB, smin, sarg):
    t = pl.program_id(1)
    zb = z_ref[...]
    z2b = z2_ref[...]

    @pl.when(t == 0)
    def _init():
        smin[...] = jnp.full((BM, NLANE), jnp.inf, jnp.float32)
        sarg[...] = jnp.zeros((BM, NLANE), jnp.float32)

    rm = smin[...]
    ra = sarg[...]
    rm, ra = _epilogue(mmB, z2b, w2b_ref[...], rm, ra, 2 * t - 1)

    mmA[...] = jax.lax.dot_general(
        zb, wa_ref[...] * 2.0, (((1,), (1,)), ((), ())),
        preferred_element_type=jnp.float32)

    rm, ra = _epilogue(mmA, z2b, w2a_ref[...], rm, ra, 2 * t)
    smin[...] = rm
    sarg[...] = ra

    mmB[...] = jax.lax.dot_general(
        zb, wb_ref[...] * 2.0, (((1,), (1,)), ((), ())),
        preferred_element_type=jnp.float32)

    @pl.when(t == GT - 1)
    def _flush():
        rmin_ref[...] = rm
        rarg_ref[...] = ra


def _final_body(rmin_ref, rarg_ref, idx_ref):
    rm = rmin_ref[...]
    col = rarg_ref[...] * 128.0 + jax.lax.broadcasted_iota(
        jnp.int32, (FBM, NLANE), 1).astype(jnp.float32)
    gmin = jnp.min(rm, axis=1, keepdims=True)
    cand = jnp.where(rm == gmin, col, jnp.float32(2**24))
    idx_ref[...] = jnp.min(cand, axis=1, keepdims=True).astype(jnp.int32)


@functools.partial(jax.jit)
def _encode(z, W, z2, w2):
    clamp = GJ - 1
    w2aug = jnp.concatenate(
        [w2, jnp.full((1, BN), jnp.inf, jnp.float32)], axis=1)
    rmin, rarg = pl.pallas_call(
        _sweep_body,
        grid=(GI, GT),
        in_specs=[
            pl.BlockSpec((BM, EMBED), lambda i, t: (i, 0)),
            pl.BlockSpec((BN, EMBED), lambda i, t: (jnp.minimum(2 * t, clamp), 0)),
            pl.BlockSpec((BN, EMBED), lambda i, t: (jnp.minimum(2 * t + 1, clamp), 0)),
            pl.BlockSpec((BM, 1), lambda i, t: (i, 0)),
            pl.BlockSpec((1, BN), lambda i, t: (0, jnp.minimum(2 * t, clamp))),
            pl.BlockSpec((1, BN), lambda i, t: (0, jnp.where(t == 0, GJ, 2 * t - 1))),
        ],
        out_specs=[
            pl.BlockSpec((BM, NLANE), lambda i, t: (i, 0)),
            pl.BlockSpec((BM, NLANE), lambda i, t: (i, 0)),
        ],
        out_shape=[
            jax.ShapeDtypeStruct((N, NLANE), jnp.float32),
            jax.ShapeDtypeStruct((N, NLANE), jnp.float32),
        ],
        scratch_shapes=[
            pltpu.VMEM((BM, BN), jnp.float32),
            pltpu.VMEM((BM, BN), jnp.float32),
            pltpu.VMEM((BM, NLANE), jnp.float32),
            pltpu.VMEM((BM, NLANE), jnp.float32),
        ],
        compiler_params=pltpu.CompilerParams(
            dimension_semantics=("parallel", "arbitrary")),
    )(z, W, W, z2, w2, w2aug)
    idx = pl.pallas_call(
        _final_body,
        grid=(N // FBM,),
        in_specs=[
            pl.BlockSpec((FBM, NLANE), lambda i: (i, 0)),
            pl.BlockSpec((FBM, NLANE), lambda i: (i, 0)),
        ],
        out_specs=pl.BlockSpec((FBM, 1), lambda i: (i, 0)),
        out_shape=jax.ShapeDtypeStruct((N, 1), jnp.int32),
        compiler_params=pltpu.CompilerParams(
            dimension_semantics=("parallel",)),
    )(rmin, rarg)
    return idx


_sc_info = plsc.get_sparse_core_info()
_NC = _sc_info.num_cores
_NS = _sc_info.num_subcores
NW = _NC * _NS
ROWS_W = N // NW
CH = 64
NCH = ROWS_W // CH
NK = EMBED // 16

_sc_mesh = plsc.VectorSubcoreMesh(core_axis_name="c", subcore_axis_name="s")


@functools.partial(
    pl.kernel,
    mesh=_sc_mesh,
    out_type=[
        jax.ShapeDtypeStruct((N, EMBED), jnp.float32),
        jax.ShapeDtypeStruct((NW, 16), jnp.float32),
    ],
    scratch_types=[
        pltpu.VMEM((CH,), jnp.int32),
        pltpu.VMEM((CH,), jnp.int32),
        pltpu.VMEM((CH, EMBED), jnp.float32),
        pltpu.VMEM((CH, EMBED), jnp.float32),
        pltpu.VMEM((CH, EMBED), jnp.float32),
        pltpu.VMEM((CH, EMBED), jnp.float32),
        pltpu.VMEM((16,), jnp.float32),
        pltpu.SemaphoreType.DMA,
        pltpu.SemaphoreType.DMA,
        pltpu.SemaphoreType.DMA,
    ],
)
def _gather_st_loss(idx_hbm, z_hbm, w_hbm, zqst_hbm, part_hbm,
                    idx0, idx1, rows0, rows1, z0, z1, acc_v,
                    gsem, zsem, osem):
    wid = lax.axis_index("s") * _NC + lax.axis_index("c")
    base = wid * ROWS_W
    bufs = ((idx0, rows0, z0), (idx1, rows1, z1))

    def start(c, slot):
        idxb, rowsb, zb = bufs[slot]
        rb = base + c * CH
        pltpu.sync_copy(idx_hbm.at[pl.ds(rb, CH)], idxb)
        g = pltpu.async_copy(w_hbm.at[idxb], rowsb, gsem)
        zc = pltpu.async_copy(z_hbm.at[pl.ds(rb, CH)], zb, zsem)
        return g, zc

    acc = jnp.zeros((16,), jnp.float32)
    inflight = {0: start(0, 0)}
    wb = {}
    for c in range(NCH):
        slot = c % 2
        if c >= 1 and (c - 1) in wb:
            wb[c - 1].wait()
        if c + 1 < NCH:
            inflight[c + 1] = start(c + 1, (c + 1) % 2)
        g, zc = inflight.pop(c)
        g.wait()
        zc.wait()
        _, rowsb, zb = bufs[slot]

        def row_body(r, acc, rowsb=rowsb, zb=zb):
            racc = jnp.zeros((16,), jnp.float32)
            for k in range(NK):
                sl = pl.ds(k * 16, 16)
                t = rowsb[r, sl] - zb[r, sl]
                rowsb[r, sl] = zb[r, sl] + t
                racc = racc + t * t
            return acc + racc

        acc = lax.fori_loop(0, CH, row_body, acc)
        wb[c] = pltpu.async_copy(
            rowsb, zqst_hbm.at[pl.ds(base + c * CH, CH)], osem)
    wb[NCH - 1].wait()
    acc_v[...] = acc
    pltpu.sync_copy(acc_v, part_hbm.at[wid])


def kernel(z, W):
    z2 = jnp.sum(z ** 2, axis=1, keepdims=True)
    w2 = jnp.sum(W ** 2, axis=1)[None, :]
    idx = _encode(z, W, z2, w2).reshape(N)
    z_q_st, parts = _gather_st_loss(idx, z, W)
    mse = parts.sum() / jnp.float32(N * EMBED)
    vq_loss = mse + 0.25 * mse
    return (z_q_st, vq_loss)
